# baseline (speedup 1.0000x reference)
import jax
import jax.numpy as jnp
from jax.experimental import pallas as pl

_B, _N, _M = 16, 2048, 2048
_NT = 512
_K = 8


def _chamfer_body(src_ref, tgtT_ref, out_ref):
    src = src_ref[0]
    t = tgtT_ref[0]
    y2 = jnp.sum(t * t, axis=0, keepdims=True)
    col_min = jnp.full((1, _M), jnp.inf, dtype=jnp.float32)
    row_total = jnp.float32(0.0)
    for i in range(_N // _NT):
        s = src[i * _NT:(i + 1) * _NT, :]
        x2 = jnp.sum(s * s, axis=1, keepdims=True)
        xy = jax.lax.dot_general(
            s, t, (((1,), (0,)), ((), ())),
            precision=jax.lax.Precision.DEFAULT,
            preferred_element_type=jnp.float32)
        d2 = (x2 + y2) - 2.0 * xy
        row_total = row_total + jnp.sum(jnp.min(d2, axis=1))
        col_min = jnp.minimum(col_min, jnp.min(d2, axis=0, keepdims=True))
    res = row_total / _N + jnp.sum(col_min) / _M
    out_ref[0] = jnp.reshape(res, (1, 1))


def kernel(src, tgt):
    pad = [(0, 0), (0, 0), (0, _K - 3)]
    srcp = jnp.pad(src, pad)
    tgtTp = jnp.transpose(jnp.pad(tgt, pad), (0, 2, 1))
    per_batch = pl.pallas_call(
        _chamfer_body,
        grid=(_B,),
        in_specs=[
            pl.BlockSpec((1, _N, _K), lambda b: (b, 0, 0)),
            pl.BlockSpec((1, _K, _M), lambda b: (b, 0, 0)),
        ],
        out_specs=pl.BlockSpec((1, 1, 1), lambda b: (b, 0, 0)),
        out_shape=jax.ShapeDtypeStruct((_B, 1, 1), jnp.float32),
    )(srcp, tgtTp)
    return jnp.sum(per_batch)
